# baseline (speedup 1.0000x reference)
import functools

import jax
import jax.numpy as jnp
from jax import lax
from jax.experimental import pallas as pl
from jax.experimental.pallas import tpu as pltpu
from jax.experimental.pallas import tpu_sc as plsc

EPS = 1e-5
NUM_TILES = 32
TILES_PER_SC = 16
CHUNK = 128
FAST_FRAC = 0.6
FAST_CID = 0

_MESH = dict(core_axis_name="c", subcore_axis_name="s")
_CPARAMS = pltpu.CompilerParams(needs_layout_passes=False,
                                use_tc_tiling_on_sc=False)


def _chunk_split(total_per_pair):
    ncf = int(round(total_per_pair * FAST_FRAC))
    return ncf, total_per_pair - ncf


@functools.lru_cache(maxsize=None)
def _make_deg_kernel(n_pad, ept):
    rpt = n_pad // TILES_PER_SC

    @functools.partial(
        pl.kernel,
        out_type=jax.ShapeDtypeStruct((2, n_pad), jnp.float32),
        mesh=plsc.VectorSubcoreMesh(**_MESH),
        compiler_params=_CPARAMS,
        scratch_types=[
            pltpu.VMEM((ept,), jnp.int32),
            pltpu.VMEM((n_pad,), jnp.float32),
            pltpu.VMEM((TILES_PER_SC, rpt), jnp.float32),
            pltpu.VMEM((rpt,), jnp.float32),
            pltpu.VMEM_SHARED((TILES_PER_SC, n_pad), jnp.float32),
        ],
    )
    def deg_kernel(dst_hbm, out_hbm, dst_v, hist_v, buf_v, red_v, stage_sh):
        cid = lax.axis_index("c")
        sid = lax.axis_index("s")
        w = cid * TILES_PER_SC + sid
        zeros = jnp.zeros((16,), jnp.float32)
        ones = jnp.ones((16,), jnp.float32)

        def zb(i, _):
            hist_v[pl.ds(i * 16, 16)] = zeros
            return 0
        lax.fori_loop(0, n_pad // 16, zb, 0)

        pltpu.sync_copy(dst_hbm.at[w], dst_v)

        def hb(i, _):
            idx = dst_v[pl.ds(i * 16, 16)]
            plsc.addupdate_scatter(hist_v, [idx], ones)
            return 0
        lax.fori_loop(0, ept // 16, hb, 0)

        pltpu.sync_copy(hist_v, stage_sh.at[sid])
        plsc.subcore_barrier()

        for r in range(TILES_PER_SC):
            pltpu.sync_copy(stage_sh.at[r, pl.ds(sid * rpt, rpt)], buf_v.at[r])

        def zr(i, _):
            red_v[pl.ds(i * 16, 16)] = zeros
            return 0
        lax.fori_loop(0, rpt // 16, zr, 0)
        for r in range(TILES_PER_SC):
            def rb(i, _, r=r):
                s = pl.ds(i * 16, 16)
                red_v[s] = red_v[s] + buf_v[r, s]
                return 0
            lax.fori_loop(0, rpt // 16, rb, 0)

        pltpu.sync_copy(red_v, out_hbm.at[cid, pl.ds(sid * rpt, rpt)])

    return deg_kernel


@functools.lru_cache(maxsize=None)
def _make_prop_kernel(n_pad, d, ncf, ncs, nparts):
    rpt = n_pad // TILES_PER_SC
    ncmax = max(ncf, ncs)

    @functools.partial(
        pl.kernel,
        out_type=[jax.ShapeDtypeStruct((2, n_pad, d), jnp.float32)
                  for _ in range(nparts)],
        mesh=plsc.VectorSubcoreMesh(**_MESH),
        compiler_params=_CPARAMS,
        scratch_types=[
            pltpu.VMEM((ncmax, CHUNK), jnp.int32),
            pltpu.VMEM((ncmax, CHUNK), jnp.int32),
            pltpu.VMEM((CHUNK, d), jnp.float32),
            pltpu.VMEM_SHARED((n_pad, d), jnp.float32),
            pltpu.SemaphoreType.DMA,
            pltpu.SemaphoreType.DMA,
        ],
    )
    def prop_kernel(*refs):
        y_hbms = refs[:nparts]
        src_hbm, dst_hbm = refs[nparts:nparts + 2]
        out_hbms = refs[nparts + 2:2 * nparts + 2]
        src_v, dst_v, rows_v, acc_sh, gsem, ssem = refs[2 * nparts + 2:]
        cid = lax.axis_index("c")
        sid = lax.axis_index("s")
        w = cid * TILES_PER_SC + sid
        nchunk = jnp.where(cid == FAST_CID, ncf, ncs)
        zeros = jnp.zeros((16,), jnp.float32)

        pltpu.sync_copy(src_hbm.at[w], src_v)
        pltpu.sync_copy(dst_hbm.at[w], dst_v)

        def zero_stripe():
            def zb(i, _):
                r = i // (d // 16)
                c = i % (d // 16)
                rows_v[r, pl.ds(c * 16, 16)] = zeros
                return 0
            lax.fori_loop(0, CHUNK * (d // 16), zb, 0)
            for j in range(rpt // CHUNK):
                pltpu.sync_copy(rows_v,
                                acc_sh.at[pl.ds(sid * rpt + j * CHUNK,
                                                CHUNK)])

        for p in range(nparts):
            zero_stripe()
            plsc.subcore_barrier()

            def body(k, _, y_hbm=y_hbms[p]):
                pltpu.async_copy(y_hbm.at[src_v.at[k]], rows_v, gsem).wait()
                pltpu.async_copy(rows_v, acc_sh.at[dst_v.at[k]],
                                 ssem, add=True).wait()
                return 0
            lax.fori_loop(0, nchunk, body, 0)

            plsc.subcore_barrier()
            pltpu.sync_copy(acc_sh.at[pl.ds(sid * rpt, rpt)],
                            out_hbms[p].at[cid, pl.ds(sid * rpt, rpt)])

    return prop_kernel


def _tc1_body(n, x_ref, w1_ref, degt_ref, y1a_ref, y1b_ref, dinv_ref):
    deg = degt_ref[:, 0:1] + degt_ref[:, 1:2] + 1.0
    dinv = lax.rsqrt(deg)[:n, :]
    xw = jnp.dot(x_ref[...], w1_ref[...], preferred_element_type=jnp.float32)
    y = xw * dinv
    half = y.shape[1] // 2
    y1a_ref[...] = y[:, :half]
    y1b_ref[...] = y[:, half:]
    dinv_ref[...] = dinv


def _tc2_body(n, acca_ref, accb_ref, y1a_ref, y1b_ref, dinv_ref, b1_ref,
              g_ref, be_ref, mu_ref, var_ref, w2_ref, y2_ref):
    dinv = dinv_ref[...]
    s1 = jnp.concatenate(
        [acca_ref[0, :n, :] + acca_ref[1, :n, :] + y1a_ref[...],
         accb_ref[0, :n, :] + accb_ref[1, :n, :] + y1b_ref[...]], axis=1)
    out1 = dinv * s1 + b1_ref[...]
    scale = lax.rsqrt(var_ref[...] + EPS) * g_ref[...]
    h = jnp.maximum(out1 * scale + (be_ref[...] - mu_ref[...] * scale), 0.0)
    y2_ref[...] = jnp.dot(h, w2_ref[...],
                          preferred_element_type=jnp.float32) * dinv


def _tc3_body(n, acc_ref, y2_ref, dinv_ref, b2_ref, cc_ref, t_ref,
              emb_ref, soft_ref):
    emb = dinv_ref[...] * (acc_ref[0, :n, :] + acc_ref[1, :n, :]
                           + y2_ref[...]) + b2_ref[...]
    emb_ref[...] = emb
    sims = lax.dot_general(emb, cc_ref[...], (((1,), (1,)), ((), ())),
                           preferred_element_type=jnp.float32)
    logits = sims / t_ref[...]
    m = jnp.max(logits, axis=1, keepdims=True)
    e = jnp.exp(logits - m)
    soft_ref[...] = e / jnp.sum(e, axis=1, keepdims=True)


def kernel(x, edge_index, W1, b1, bn_gamma, bn_beta, bn_mean, bn_var,
           W2, b2, cluster_centers, temperature):
    n, d_in = x.shape
    d_h = W1.shape[1]
    d_emb = W2.shape[1]
    e = edge_index.shape[1]
    k = cluster_centers.shape[0]

    n_pad = -(-n // (16 * TILES_PER_SC)) * (16 * TILES_PER_SC)
    nchunk_pair = 2 * (-(-e // (NUM_TILES * CHUNK)))
    ncf, ncs = _chunk_split(nchunk_pair)
    e_pad = TILES_PER_SC * nchunk_pair * CHUNK

    src = edge_index[0]
    dst = edge_index[1]
    src_flat = jnp.concatenate([src, jnp.zeros((e_pad - e,), jnp.int32)])
    dst_flat = jnp.concatenate([dst, jnp.full((e_pad - e,), n, jnp.int32)])

    def to_tiles(flat, fill):
        ncmax = max(ncf, ncs)
        chunks = flat.reshape(-1, CHUNK)
        nf = TILES_PER_SC * ncf
        fast = chunks[:nf].reshape(TILES_PER_SC, ncf, CHUNK)
        slow = chunks[nf:].reshape(TILES_PER_SC, ncs, CHUNK)
        if ncs < ncmax:
            pad = jnp.full((TILES_PER_SC, ncmax - ncs, CHUNK), fill,
                           jnp.int32)
            slow = jnp.concatenate([slow, pad], axis=1)
        if ncf < ncmax:
            pad = jnp.full((TILES_PER_SC, ncmax - ncf, CHUNK), fill,
                           jnp.int32)
            fast = jnp.concatenate([fast, pad], axis=1)
        both = (fast, slow) if FAST_CID == 0 else (slow, fast)
        return jnp.concatenate(both, axis=0)

    src_p = to_tiles(src_flat, 0)
    dst_p = to_tiles(dst_flat, n)

    ept_deg = e_pad // NUM_TILES
    deg_parts = _make_deg_kernel(n_pad, ept_deg)(
        dst_flat.reshape(NUM_TILES, ept_deg))
    degt = deg_parts.T

    half = d_h // 2
    y1a, y1b, dinv = pl.pallas_call(
        functools.partial(_tc1_body, n),
        out_shape=[jax.ShapeDtypeStruct((n, half), jnp.float32),
                   jax.ShapeDtypeStruct((n, half), jnp.float32),
                   jax.ShapeDtypeStruct((n, 1), jnp.float32)],
    )(x, W1, degt)

    acc1a, acc1b = _make_prop_kernel(n_pad, half, ncf, ncs, 2)(
        y1a, y1b, src_p, dst_p)

    y2 = pl.pallas_call(
        functools.partial(_tc2_body, n),
        out_shape=jax.ShapeDtypeStruct((n, d_emb), jnp.float32),
    )(acc1a, acc1b, y1a, y1b, dinv, b1.reshape(1, d_h),
      bn_gamma.reshape(1, d_h), bn_beta.reshape(1, d_h),
      bn_mean.reshape(1, d_h), bn_var.reshape(1, d_h), W2)

    acc2, = _make_prop_kernel(n_pad, d_emb, ncf, ncs, 1)(y2, src_p, dst_p)

    emb, soft = pl.pallas_call(
        functools.partial(_tc3_body, n),
        out_shape=[jax.ShapeDtypeStruct((n, d_emb), jnp.float32),
                   jax.ShapeDtypeStruct((n, k), jnp.float32)],
    )(acc2, y2, dinv, b2.reshape(1, d_emb), cluster_centers,
      temperature.reshape(1, 1))

    return emb, soft


# speedup vs baseline: 1.0170x; 1.0170x over previous
import functools

import jax
import jax.numpy as jnp
from jax import lax
from jax.experimental import pallas as pl
from jax.experimental.pallas import tpu as pltpu
from jax.experimental.pallas import tpu_sc as plsc

EPS = 1e-5
NUM_TILES = 32
TILES_PER_SC = 16
CHUNK = 128
FAST_FRAC = 0.63
FAST_CID = 0

_MESH = dict(core_axis_name="c", subcore_axis_name="s")
_CPARAMS = pltpu.CompilerParams(needs_layout_passes=False,
                                use_tc_tiling_on_sc=False)


def _chunk_split(total_per_pair):
    ncf = int(round(total_per_pair * FAST_FRAC))
    return ncf, total_per_pair - ncf


@functools.lru_cache(maxsize=None)
def _make_deg_kernel(n_pad, ept):
    rpt = n_pad // TILES_PER_SC

    @functools.partial(
        pl.kernel,
        out_type=jax.ShapeDtypeStruct((2, n_pad), jnp.float32),
        mesh=plsc.VectorSubcoreMesh(**_MESH),
        compiler_params=_CPARAMS,
        scratch_types=[
            pltpu.VMEM((ept,), jnp.int32),
            pltpu.VMEM((n_pad,), jnp.float32),
            pltpu.VMEM((TILES_PER_SC, rpt), jnp.float32),
            pltpu.VMEM((rpt,), jnp.float32),
            pltpu.VMEM_SHARED((TILES_PER_SC, n_pad), jnp.float32),
        ],
    )
    def deg_kernel(dst_hbm, out_hbm, dst_v, hist_v, buf_v, red_v, stage_sh):
        cid = lax.axis_index("c")
        sid = lax.axis_index("s")
        w = cid * TILES_PER_SC + sid
        zeros = jnp.zeros((16,), jnp.float32)
        ones = jnp.ones((16,), jnp.float32)

        def zb(i, _):
            hist_v[pl.ds(i * 16, 16)] = zeros
            return 0
        lax.fori_loop(0, n_pad // 16, zb, 0)

        pltpu.sync_copy(dst_hbm.at[w], dst_v)

        def hb(i, _):
            idx = dst_v[pl.ds(i * 16, 16)]
            plsc.addupdate_scatter(hist_v, [idx], ones)
            return 0
        lax.fori_loop(0, ept // 16, hb, 0)

        pltpu.sync_copy(hist_v, stage_sh.at[sid])
        plsc.subcore_barrier()

        for r in range(TILES_PER_SC):
            pltpu.sync_copy(stage_sh.at[r, pl.ds(sid * rpt, rpt)], buf_v.at[r])

        def zr(i, _):
            red_v[pl.ds(i * 16, 16)] = zeros
            return 0
        lax.fori_loop(0, rpt // 16, zr, 0)
        for r in range(TILES_PER_SC):
            def rb(i, _, r=r):
                s = pl.ds(i * 16, 16)
                red_v[s] = red_v[s] + buf_v[r, s]
                return 0
            lax.fori_loop(0, rpt // 16, rb, 0)

        pltpu.sync_copy(red_v, out_hbm.at[cid, pl.ds(sid * rpt, rpt)])

    return deg_kernel


@functools.lru_cache(maxsize=None)
def _make_prop_kernel(n_pad, d, ncf, ncs, nparts):
    rpt = n_pad // TILES_PER_SC
    ncmax = max(ncf, ncs)

    @functools.partial(
        pl.kernel,
        out_type=[jax.ShapeDtypeStruct((2, n_pad, d), jnp.float32)
                  for _ in range(nparts)],
        mesh=plsc.VectorSubcoreMesh(**_MESH),
        compiler_params=_CPARAMS,
        scratch_types=[
            pltpu.VMEM((ncmax, CHUNK), jnp.int32),
            pltpu.VMEM((ncmax, CHUNK), jnp.int32),
            pltpu.VMEM((CHUNK, d), jnp.float32),
            pltpu.VMEM_SHARED((n_pad, d), jnp.float32),
            pltpu.SemaphoreType.DMA,
            pltpu.SemaphoreType.DMA,
        ],
    )
    def prop_kernel(*refs):
        y_hbms = refs[:nparts]
        src_hbm, dst_hbm = refs[nparts:nparts + 2]
        out_hbms = refs[nparts + 2:2 * nparts + 2]
        src_v, dst_v, rows_v, acc_sh, gsem, ssem = refs[2 * nparts + 2:]
        cid = lax.axis_index("c")
        sid = lax.axis_index("s")
        w = cid * TILES_PER_SC + sid
        nchunk = jnp.where(cid == FAST_CID, ncf, ncs)
        zeros = jnp.zeros((16,), jnp.float32)

        pltpu.sync_copy(src_hbm.at[w], src_v)
        pltpu.sync_copy(dst_hbm.at[w], dst_v)

        def zero_stripe():
            def zb(i, _):
                r = i // (d // 16)
                c = i % (d // 16)
                rows_v[r, pl.ds(c * 16, 16)] = zeros
                return 0
            lax.fori_loop(0, CHUNK * (d // 16), zb, 0)
            for j in range(rpt // CHUNK):
                pltpu.sync_copy(rows_v,
                                acc_sh.at[pl.ds(sid * rpt + j * CHUNK,
                                                CHUNK)])

        for p in range(nparts):
            zero_stripe()
            plsc.subcore_barrier()

            def body(k, _, y_hbm=y_hbms[p]):
                pltpu.async_copy(y_hbm.at[src_v.at[k]], rows_v, gsem).wait()
                pltpu.async_copy(rows_v, acc_sh.at[dst_v.at[k]],
                                 ssem, add=True).wait()
                return 0
            lax.fori_loop(0, nchunk, body, 0)

            plsc.subcore_barrier()
            pltpu.sync_copy(acc_sh.at[pl.ds(sid * rpt, rpt)],
                            out_hbms[p].at[cid, pl.ds(sid * rpt, rpt)])

    return prop_kernel


def _tc1_body(n, x_ref, w1_ref, degt_ref, y1a_ref, y1b_ref, dinv_ref):
    deg = degt_ref[:, 0:1] + degt_ref[:, 1:2] + 1.0
    dinv = lax.rsqrt(deg)[:n, :]
    xw = jnp.dot(x_ref[...], w1_ref[...], preferred_element_type=jnp.float32)
    y = xw * dinv
    half = y.shape[1] // 2
    y1a_ref[...] = y[:, :half]
    y1b_ref[...] = y[:, half:]
    dinv_ref[...] = dinv


def _tc2_body(n, acca_ref, accb_ref, y1a_ref, y1b_ref, dinv_ref, b1_ref,
              g_ref, be_ref, mu_ref, var_ref, w2_ref, y2_ref):
    dinv = dinv_ref[...]
    s1 = jnp.concatenate(
        [acca_ref[0, :n, :] + acca_ref[1, :n, :] + y1a_ref[...],
         accb_ref[0, :n, :] + accb_ref[1, :n, :] + y1b_ref[...]], axis=1)
    out1 = dinv * s1 + b1_ref[...]
    scale = lax.rsqrt(var_ref[...] + EPS) * g_ref[...]
    h = jnp.maximum(out1 * scale + (be_ref[...] - mu_ref[...] * scale), 0.0)
    y2_ref[...] = jnp.dot(h, w2_ref[...],
                          preferred_element_type=jnp.float32) * dinv


def _tc3_body(n, acc_ref, y2_ref, dinv_ref, b2_ref, cc_ref, t_ref,
              emb_ref, soft_ref):
    emb = dinv_ref[...] * (acc_ref[0, :n, :] + acc_ref[1, :n, :]
                           + y2_ref[...]) + b2_ref[...]
    emb_ref[...] = emb
    sims = lax.dot_general(emb, cc_ref[...], (((1,), (1,)), ((), ())),
                           preferred_element_type=jnp.float32)
    logits = sims / t_ref[...]
    m = jnp.max(logits, axis=1, keepdims=True)
    e = jnp.exp(logits - m)
    soft_ref[...] = e / jnp.sum(e, axis=1, keepdims=True)


def kernel(x, edge_index, W1, b1, bn_gamma, bn_beta, bn_mean, bn_var,
           W2, b2, cluster_centers, temperature):
    n, d_in = x.shape
    d_h = W1.shape[1]
    d_emb = W2.shape[1]
    e = edge_index.shape[1]
    k = cluster_centers.shape[0]

    n_pad = -(-n // (16 * TILES_PER_SC)) * (16 * TILES_PER_SC)
    nchunk_pair = 2 * (-(-e // (NUM_TILES * CHUNK)))
    ncf, ncs = _chunk_split(nchunk_pair)
    e_pad = TILES_PER_SC * nchunk_pair * CHUNK

    src = edge_index[0]
    dst = edge_index[1]
    src_flat = jnp.concatenate([src, jnp.zeros((e_pad - e,), jnp.int32)])
    dst_flat = jnp.concatenate([dst, jnp.full((e_pad - e,), n, jnp.int32)])

    def to_tiles(flat, fill):
        ncmax = max(ncf, ncs)
        chunks = flat.reshape(-1, CHUNK)
        nf = TILES_PER_SC * ncf
        fast = chunks[:nf].reshape(TILES_PER_SC, ncf, CHUNK)
        slow = chunks[nf:].reshape(TILES_PER_SC, ncs, CHUNK)
        if ncs < ncmax:
            pad = jnp.full((TILES_PER_SC, ncmax - ncs, CHUNK), fill,
                           jnp.int32)
            slow = jnp.concatenate([slow, pad], axis=1)
        if ncf < ncmax:
            pad = jnp.full((TILES_PER_SC, ncmax - ncf, CHUNK), fill,
                           jnp.int32)
            fast = jnp.concatenate([fast, pad], axis=1)
        both = (fast, slow) if FAST_CID == 0 else (slow, fast)
        return jnp.concatenate(both, axis=0)

    src_p = to_tiles(src_flat, 0)
    dst_p = to_tiles(dst_flat, n)

    ept_deg = e_pad // NUM_TILES
    deg_parts = _make_deg_kernel(n_pad, ept_deg)(
        dst_flat.reshape(NUM_TILES, ept_deg))
    degt = deg_parts.T

    half = d_h // 2
    y1a, y1b, dinv = pl.pallas_call(
        functools.partial(_tc1_body, n),
        out_shape=[jax.ShapeDtypeStruct((n, half), jnp.float32),
                   jax.ShapeDtypeStruct((n, half), jnp.float32),
                   jax.ShapeDtypeStruct((n, 1), jnp.float32)],
    )(x, W1, degt)

    prop = _make_prop_kernel(n_pad, half, ncf, ncs, 1)
    acc1a, = prop(y1a, src_p, dst_p)
    acc1b, = prop(y1b, src_p, dst_p)

    y2 = pl.pallas_call(
        functools.partial(_tc2_body, n),
        out_shape=jax.ShapeDtypeStruct((n, d_emb), jnp.float32),
    )(acc1a, acc1b, y1a, y1b, dinv, b1.reshape(1, d_h),
      bn_gamma.reshape(1, d_h), bn_beta.reshape(1, d_h),
      bn_mean.reshape(1, d_h), bn_var.reshape(1, d_h), W2)

    acc2, = _make_prop_kernel(n_pad, d_emb, ncf, ncs, 1)(y2, src_p, dst_p)

    emb, soft = pl.pallas_call(
        functools.partial(_tc3_body, n),
        out_shape=[jax.ShapeDtypeStruct((n, d_emb), jnp.float32),
                   jax.ShapeDtypeStruct((n, k), jnp.float32)],
    )(acc2, y2, dinv, b2.reshape(1, d_emb), cluster_centers,
      temperature.reshape(1, 1))

    return emb, soft
